# baseline (speedup 1.0000x reference)
import jax
import jax.numpy as jnp
import numpy as np
from jax.experimental import pallas as pl
from jax.experimental.pallas import tpu as pltpu

B, N, E, C, NG, L = 2, 8192, 256, 4, 8, 2
CE = E // C
M = N // NG
G = 128
NB = M // G
H = NG
HR = H * NG
MOM = 0.9
NSTEP = L * NG
CINV = [0.1 / MOM ** (k + 1) for k in range(NSTEP)]
APK = [MOM ** k for k in range(NSTEP)]
BZK = [MOM ** (2 * k) for k in range(NSTEP)]
FINAL = MOM ** NSTEP

_SEG = np.arange(E) // CE
_SEL = ((np.arange(2 * C)[:, None] == _SEG[None, :])
        | ((np.arange(2 * C)[:, None] - C) == _SEG[None, :])).astype(np.float32)
_OH = (_SEG[:, None] == np.arange(C)[None, :]).astype(np.float32)


def _make_bdk(weights, alpha):
    w1 = weights[:, :E]
    w2 = weights[:, E:]
    oh = jnp.asarray(_OH)
    bd1 = w1[:, :, None] * oh[None]
    bd2 = w2[:, :, None] * oh[None]
    bd = jnp.concatenate([bd1, bd2], axis=-1)
    bd_steps = jnp.repeat(bd, NG, axis=0)
    scale = (1.0 - alpha) * jnp.asarray(BZK, jnp.float32)
    return bd_steps * scale[:, None, None]


def _body(x_ref, xt_ref, xh_ref, xa_ref, xat_ref, xah_ref,
          bdk_ref, sel_ref, alpha_ref, y_ref, ya_ref):
    alpha = alpha_ref[0, 0]
    sel = sel_ref[...]

    def layer(u, v, k0):
        rows = u.shape[0]
        m = rows // NG
        for j in range(NG):
            k = k0 + j
            u3 = u.reshape(m, NG, E)
            ujb = jnp.broadcast_to(u3[:, j:j + 1, :], (m, NG, E)).reshape(rows, E)
            r_p = jnp.dot(u, bdk_ref[k][:, :C], preferred_element_type=jnp.float32)
            r_q = jnp.dot(ujb, bdk_ref[k][:, C:], preferred_element_type=jnp.float32)
            rc = jnp.concatenate([r_p, r_q], axis=1)
            wb = jnp.dot(rc, sel, preferred_element_type=jnp.float32)
            t = (alpha * APK[k]) * u + wb * ujb
            fv = jnp.tanh(t)
            v = v + CINV[k] * fv
            u = u + 0.1 * v
        return u, v

    xi = jnp.concatenate([xt_ref[0, 0], x_ref[0], xh_ref[0, 0]], axis=0)
    xa = jnp.concatenate([xat_ref[0, 0], xa_ref[0], xah_ref[0, 0]], axis=0)
    u, v = layer(xi, xa, 0)

    def skew_fwd(a):
        a3 = a.reshape(G + 2 * H, NG, E)
        cols = [a3[o:o + G + H, o, :][:, None, :] for o in range(NG)]
        return jnp.concatenate(cols, axis=1).reshape((G + H) * NG, E)

    u, v = skew_fwd(u), skew_fwd(v)
    u, v = layer(u, v, NG)

    def skew_inv(a):
        a3 = a.reshape(G + H, NG, E)
        cols = [(FINAL * a3[H - o:H - o + G, o, :])[:, None, :] for o in range(NG)]
        return jnp.concatenate(cols, axis=1).reshape(G * NG, E)

    y_ref[0] = skew_inv(u)
    ya_ref[0] = skew_inv(v)


def kernel(x, xa, weights, alpha, n, C_):
    del n, C_
    alpha_f = jnp.asarray(alpha, jnp.float32)
    bdk = _make_bdk(weights.astype(jnp.float32), alpha_f)
    alpha_arr = alpha_f.reshape(1, 1)
    sel = jnp.asarray(_SEL)

    x4 = x.reshape(B, NB, G * NG, E)
    xa4 = xa.reshape(B, NB, G * NG, E)
    xh, xt = x4[:, :, :HR, :], x4[:, :, -HR:, :]
    xah, xat = xa4[:, :, :HR, :], xa4[:, :, -HR:, :]

    blk = pl.BlockSpec((1, G * NG, E), lambda b, g: (b, g, 0))
    prev_tail = pl.BlockSpec((1, 1, HR, E), lambda b, g: (b, (g - 1) % NB, 0, 0))
    next_head = pl.BlockSpec((1, 1, HR, E), lambda b, g: (b, (g + 1) % NB, 0, 0))

    y, ya = pl.pallas_call(
        _body,
        grid=(B, NB),
        in_specs=[
            blk, prev_tail, next_head,
            blk, prev_tail, next_head,
            pl.BlockSpec((NSTEP, E, 2 * C), lambda b, g: (0, 0, 0)),
            pl.BlockSpec((2 * C, E), lambda b, g: (0, 0)),
            pl.BlockSpec(memory_space=pltpu.SMEM),
        ],
        out_specs=[blk, blk],
        out_shape=[
            jax.ShapeDtypeStruct((B, N, E), jnp.float32),
            jax.ShapeDtypeStruct((B, N, E), jnp.float32),
        ],
    )(x, xt, xh, xa, xat, xah, bdk, sel, alpha_arr)
    return y, ya


# speedup vs baseline: 1.3771x; 1.3771x over previous
import jax
import jax.numpy as jnp
import numpy as np
from jax.experimental import pallas as pl
from jax.experimental.pallas import tpu as pltpu

B, N, E, C, NG, L = 2, 8192, 256, 4, 8, 2
CE = E // C
M = N // NG
G = 256
NB = M // G
H = NG
HR = H * NG
MOM = 0.9
NSTEP = L * NG
CINV = [0.1 / MOM ** (k + 1) for k in range(NSTEP)]
APK = [MOM ** k for k in range(NSTEP)]
BZK = [MOM ** (2 * k) for k in range(NSTEP)]
FINAL = MOM ** NSTEP

_SEG = np.arange(E) // CE
_SEL = ((np.arange(2 * C)[:, None] == _SEG[None, :])
        | ((np.arange(2 * C)[:, None] - C) == _SEG[None, :])).astype(np.float32)
_OH = (_SEG[:, None] == np.arange(C)[None, :]).astype(np.float32)


def _make_bdk(weights, alpha):
    w1 = weights[:, :E]
    w2 = weights[:, E:]
    oh = jnp.asarray(_OH)
    bd1 = w1[:, :, None] * oh[None]
    bd2 = w2[:, :, None] * oh[None]
    bd = jnp.concatenate([bd1, bd2], axis=-1)
    bd_steps = jnp.repeat(bd, NG, axis=0)
    scale = (1.0 - alpha) * jnp.asarray(BZK, jnp.float32)
    return bd_steps * scale[:, None, None]


def _body(x_ref, xt_ref, xh_ref, xa_ref, xat_ref, xah_ref,
          bdk_ref, sel_ref, alpha_ref, y_ref, ya_ref):
    alpha = alpha_ref[0, 0]
    sel = sel_ref[...]

    def layer(u, v, k0):
        rows = u.shape[0]
        m = rows // NG
        for j in range(NG):
            k = k0 + j
            r = jnp.dot(u, bdk_ref[k], preferred_element_type=jnp.float32)
            r3 = r.reshape(m, NG, 2 * C)
            qrow = jnp.broadcast_to(r3[:, j:j + 1, C:], (m, NG, C))
            rc = jnp.concatenate([r3[:, :, :C], qrow], axis=2).reshape(rows, 2 * C)
            wb = jnp.dot(rc, sel, preferred_element_type=jnp.float32)
            u3 = u.reshape(m, NG, E)
            ujb = jnp.broadcast_to(u3[:, j:j + 1, :], (m, NG, E)).reshape(rows, E)
            t = (alpha * APK[k]) * u + wb * ujb
            fv = jnp.tanh(t)
            v = v + CINV[k] * fv
            u = u + 0.1 * v
        return u, v

    xi = jnp.concatenate([xt_ref[0, 0], x_ref[0], xh_ref[0, 0]], axis=0)
    xa = jnp.concatenate([xat_ref[0, 0], xa_ref[0], xah_ref[0, 0]], axis=0)
    u, v = layer(xi, xa, 0)

    def skew_fwd(a):
        a3 = a.reshape(G + 2 * H, NG, E)
        cols = [a3[o:o + G + H, o, :][:, None, :] for o in range(NG)]
        return jnp.concatenate(cols, axis=1).reshape((G + H) * NG, E)

    u, v = skew_fwd(u), skew_fwd(v)
    u, v = layer(u, v, NG)

    def skew_inv(a):
        a3 = a.reshape(G + H, NG, E)
        cols = [(FINAL * a3[H - o:H - o + G, o, :])[:, None, :] for o in range(NG)]
        return jnp.concatenate(cols, axis=1).reshape(G * NG, E)

    y_ref[0] = skew_inv(u)
    ya_ref[0] = skew_inv(v)


def kernel(x, xa, weights, alpha, n, C_):
    del n, C_
    alpha_f = jnp.asarray(alpha, jnp.float32)
    bdk = _make_bdk(weights.astype(jnp.float32), alpha_f)
    alpha_arr = alpha_f.reshape(1, 1)
    sel = jnp.asarray(_SEL)

    x4 = x.reshape(B, NB, G * NG, E)
    xa4 = xa.reshape(B, NB, G * NG, E)
    xh, xt = x4[:, :, :HR, :], x4[:, :, -HR:, :]
    xah, xat = xa4[:, :, :HR, :], xa4[:, :, -HR:, :]

    blk = pl.BlockSpec((1, G * NG, E), lambda b, g: (b, g, 0))
    prev_tail = pl.BlockSpec((1, 1, HR, E), lambda b, g: (b, (g - 1) % NB, 0, 0))
    next_head = pl.BlockSpec((1, 1, HR, E), lambda b, g: (b, (g + 1) % NB, 0, 0))

    y, ya = pl.pallas_call(
        _body,
        grid=(B, NB),
        in_specs=[
            blk, prev_tail, next_head,
            blk, prev_tail, next_head,
            pl.BlockSpec((NSTEP, E, 2 * C), lambda b, g: (0, 0, 0)),
            pl.BlockSpec((2 * C, E), lambda b, g: (0, 0)),
            pl.BlockSpec(memory_space=pltpu.SMEM),
        ],
        out_specs=[blk, blk],
        out_shape=[
            jax.ShapeDtypeStruct((B, N, E), jnp.float32),
            jax.ShapeDtypeStruct((B, N, E), jnp.float32),
        ],
    )(x, xt, xh, xa, xat, xah, bdk, sel, alpha_arr)
    return y, ya


# speedup vs baseline: 1.3923x; 1.0110x over previous
import jax
import jax.numpy as jnp
import numpy as np
from jax.experimental import pallas as pl
from jax.experimental.pallas import tpu as pltpu

B, N, E, C, NG, L = 2, 8192, 256, 4, 8, 2
CE = E // C
M = N // NG
G = 512
NB = M // G
H = NG
HR = H * NG
MOM = 0.9
NSTEP = L * NG
CINV = [0.1 / MOM ** (k + 1) for k in range(NSTEP)]
APK = [MOM ** k for k in range(NSTEP)]
BZK = [MOM ** (2 * k) for k in range(NSTEP)]
FINAL = MOM ** NSTEP

_SEG = np.arange(E) // CE
_SEL = ((np.arange(2 * C)[:, None] == _SEG[None, :])
        | ((np.arange(2 * C)[:, None] - C) == _SEG[None, :])).astype(np.float32)
_OH = (_SEG[:, None] == np.arange(C)[None, :]).astype(np.float32)


def _make_bdk(weights, alpha):
    w1 = weights[:, :E]
    w2 = weights[:, E:]
    oh = jnp.asarray(_OH)
    bd1 = w1[:, :, None] * oh[None]
    bd2 = w2[:, :, None] * oh[None]
    bd = jnp.concatenate([bd1, bd2], axis=-1)
    bd_steps = jnp.repeat(bd, NG, axis=0)
    scale = (1.0 - alpha) * jnp.asarray(BZK, jnp.float32)
    return bd_steps * scale[:, None, None]


def _body(x_ref, xt_ref, xh_ref, xa_ref, xat_ref, xah_ref,
          bdk_ref, sel_ref, alpha_ref, y_ref, ya_ref):
    alpha = alpha_ref[0, 0]
    sel = sel_ref[...]

    def layer(u, v, k0):
        rows = u.shape[0]
        m = rows // NG
        for j in range(NG):
            k = k0 + j
            r = jnp.dot(u, bdk_ref[k], preferred_element_type=jnp.float32)
            r3 = r.reshape(m, NG, 2 * C)
            qrow = jnp.broadcast_to(r3[:, j:j + 1, C:], (m, NG, C))
            rc = jnp.concatenate([r3[:, :, :C], qrow], axis=2).reshape(rows, 2 * C)
            wb = jnp.dot(rc, sel, preferred_element_type=jnp.float32)
            u3 = u.reshape(m, NG, E)
            ujb = jnp.broadcast_to(u3[:, j:j + 1, :], (m, NG, E)).reshape(rows, E)
            t = (alpha * APK[k]) * u + wb * ujb
            fv = jnp.tanh(t)
            v = v + CINV[k] * fv
            u = u + 0.1 * v
        return u, v

    xi = jnp.concatenate([xt_ref[0, 0], x_ref[0], xh_ref[0, 0]], axis=0)
    xa = jnp.concatenate([xat_ref[0, 0], xa_ref[0], xah_ref[0, 0]], axis=0)
    u, v = layer(xi, xa, 0)

    def skew_fwd(a):
        a3 = a.reshape(G + 2 * H, NG, E)
        cols = [a3[o:o + G + H, o, :][:, None, :] for o in range(NG)]
        return jnp.concatenate(cols, axis=1).reshape((G + H) * NG, E)

    u, v = skew_fwd(u), skew_fwd(v)
    u, v = layer(u, v, NG)

    def skew_inv(a):
        a3 = a.reshape(G + H, NG, E)
        cols = [(FINAL * a3[H - o:H - o + G, o, :])[:, None, :] for o in range(NG)]
        return jnp.concatenate(cols, axis=1).reshape(G * NG, E)

    y_ref[0] = skew_inv(u)
    ya_ref[0] = skew_inv(v)


def kernel(x, xa, weights, alpha, n, C_):
    del n, C_
    alpha_f = jnp.asarray(alpha, jnp.float32)
    bdk = _make_bdk(weights.astype(jnp.float32), alpha_f)
    alpha_arr = alpha_f.reshape(1, 1)
    sel = jnp.asarray(_SEL)

    x4 = x.reshape(B, NB, G * NG, E)
    xa4 = xa.reshape(B, NB, G * NG, E)
    xh, xt = x4[:, :, :HR, :], x4[:, :, -HR:, :]
    xah, xat = xa4[:, :, :HR, :], xa4[:, :, -HR:, :]

    blk = pl.BlockSpec((1, G * NG, E), lambda b, g: (b, g, 0))
    prev_tail = pl.BlockSpec((1, 1, HR, E), lambda b, g: (b, (g - 1) % NB, 0, 0))
    next_head = pl.BlockSpec((1, 1, HR, E), lambda b, g: (b, (g + 1) % NB, 0, 0))

    y, ya = pl.pallas_call(
        _body,
        grid=(B, NB),
        in_specs=[
            blk, prev_tail, next_head,
            blk, prev_tail, next_head,
            pl.BlockSpec((NSTEP, E, 2 * C), lambda b, g: (0, 0, 0)),
            pl.BlockSpec((2 * C, E), lambda b, g: (0, 0)),
            pl.BlockSpec(memory_space=pltpu.SMEM),
        ],
        out_specs=[blk, blk],
        out_shape=[
            jax.ShapeDtypeStruct((B, N, E), jnp.float32),
            jax.ShapeDtypeStruct((B, N, E), jnp.float32),
        ],
    )(x, xt, xh, xa, xat, xah, bdk, sel, alpha_arr)
    return y, ya


# speedup vs baseline: 1.4980x; 1.0759x over previous
import jax
import jax.numpy as jnp
import numpy as np
from jax.experimental import pallas as pl
from jax.experimental.pallas import tpu as pltpu

B, N, E, C, NG, L = 2, 8192, 256, 4, 8, 2
CE = E // C
M = N // NG
G = 512
NB = M // G
H = NG
HR = H * NG
MOM = 0.9
NSTEP = L * NG
CINV = [0.01 / MOM ** (k + 1) for k in range(NSTEP)]
APK = [MOM ** k for k in range(NSTEP)]
BZK = [MOM ** (2 * k) for k in range(NSTEP)]
FINAL = MOM ** NSTEP

_SEG = np.arange(E) // CE
_SEL = ((np.arange(2 * C)[:, None] == _SEG[None, :])
        | ((np.arange(2 * C)[:, None] - C) == _SEG[None, :])).astype(np.float32)
_OH = (_SEG[:, None] == np.arange(C)[None, :]).astype(np.float32)


def _make_bdk(weights, alpha):
    w1 = weights[:, :E]
    w2 = weights[:, E:]
    oh = jnp.asarray(_OH)
    bd1 = w1[:, :, None] * oh[None]
    bd2 = w2[:, :, None] * oh[None]
    bd = jnp.concatenate([bd1, bd2], axis=-1)
    bd_steps = jnp.repeat(bd, NG, axis=0)
    scale = (1.0 - alpha) * jnp.asarray(BZK, jnp.float32)
    return bd_steps * scale[:, None, None]


def _body(x_ref, xt_ref, xh_ref, xa_ref, xat_ref, xah_ref,
          bdk_ref, sel_ref, alpha_ref, y_ref, ya_ref):
    alpha = alpha_ref[0, 0]
    sel = sel_ref[...]

    def layer(u, v, k0):
        rows = u.shape[0]
        m = rows // NG
        for j in range(NG):
            k = k0 + j
            r = jnp.dot(u, bdk_ref[k], preferred_element_type=jnp.float32)
            r3 = r.reshape(m, NG, 2 * C)
            qrow = jnp.broadcast_to(r3[:, j:j + 1, C:], (m, NG, C))
            rc = jnp.concatenate([r3[:, :, :C], qrow], axis=2).reshape(rows, 2 * C)
            wb = jnp.dot(rc, sel, preferred_element_type=jnp.float32)
            u3 = u.reshape(m, NG, E)
            ujb = jnp.broadcast_to(u3[:, j:j + 1, :], (m, NG, E)).reshape(rows, E)
            t = (alpha * APK[k]) * u + wb * ujb
            fv = jnp.tanh(t)
            v = v + CINV[k] * fv
            u = u + v
        return u, v

    xi = jnp.concatenate([xt_ref[0, 0], x_ref[0], xh_ref[0, 0]], axis=0)
    xa = jnp.concatenate([xat_ref[0, 0], xa_ref[0], xah_ref[0, 0]], axis=0)
    u, v = layer(xi, 0.1 * xa, 0)

    def skew_fwd(a):
        a3 = a.reshape(G + 2 * H, NG, E)
        cols = [a3[o:o + G + H, o, :][:, None, :] for o in range(NG)]
        return jnp.concatenate(cols, axis=1).reshape((G + H) * NG, E)

    u, v = skew_fwd(u), skew_fwd(v)
    u, v = layer(u, v, NG)

    def skew_inv(a, scale):
        a3 = a.reshape(G + H, NG, E)
        cols = [(scale * a3[H - o:H - o + G, o, :])[:, None, :] for o in range(NG)]
        return jnp.concatenate(cols, axis=1).reshape(G * NG, E)

    y_ref[0] = skew_inv(u, FINAL)
    ya_ref[0] = skew_inv(v, 10.0 * FINAL)


def kernel(x, xa, weights, alpha, n, C_):
    del n, C_
    alpha_f = jnp.asarray(alpha, jnp.float32)
    bdk = _make_bdk(weights.astype(jnp.float32), alpha_f)
    alpha_arr = alpha_f.reshape(1, 1)
    sel = jnp.asarray(_SEL)

    x4 = x.reshape(B, NB, G * NG, E)
    xa4 = xa.reshape(B, NB, G * NG, E)
    xh, xt = x4[:, :, :HR, :], x4[:, :, -HR:, :]
    xah, xat = xa4[:, :, :HR, :], xa4[:, :, -HR:, :]

    blk = pl.BlockSpec((1, G * NG, E), lambda b, g: (b, g, 0))
    prev_tail = pl.BlockSpec((1, 1, HR, E), lambda b, g: (b, (g - 1) % NB, 0, 0))
    next_head = pl.BlockSpec((1, 1, HR, E), lambda b, g: (b, (g + 1) % NB, 0, 0))

    y, ya = pl.pallas_call(
        _body,
        grid=(B, NB),
        in_specs=[
            blk, prev_tail, next_head,
            blk, prev_tail, next_head,
            pl.BlockSpec((NSTEP, E, 2 * C), lambda b, g: (0, 0, 0)),
            pl.BlockSpec((2 * C, E), lambda b, g: (0, 0)),
            pl.BlockSpec(memory_space=pltpu.SMEM),
        ],
        out_specs=[blk, blk],
        out_shape=[
            jax.ShapeDtypeStruct((B, N, E), jnp.float32),
            jax.ShapeDtypeStruct((B, N, E), jnp.float32),
        ],
    )(x, xt, xh, xa, xat, xah, bdk, sel, alpha_arr)
    return y, ya
